# baseline (speedup 1.0000x reference)
import functools

import jax
import jax.numpy as jnp
from jax import lax
from jax.experimental import pallas as pl
from jax.experimental.pallas import tpu as pltpu
from jax.experimental.pallas import tpu_sc as plsc

_VOCAB = 1000000
_HID = 64
_B = 16384
_CTX = 50

_NC = 2
_NS = 16
_L = 16
_NW = _NC * _NS
_RW = _B // _NW
_PAIR = 2
_CHI = 104
_NCHK = _RW // _PAIR
_REAL = _PAIR * _CTX
_NBUF = 4


_SC_KERNEL_KWARGS = dict(
    mesh=plsc.VectorSubcoreMesh(core_axis_name="c", subcore_axis_name="s"),
    out_type=jax.ShapeDtypeStruct((_B,), jnp.float32),
    scratch_types=[
        pltpu.VMEM((_NCHK, _CHI), jnp.int32),
        pltpu.VMEM((_NBUF, _CHI, _HID), jnp.float32),
        pltpu.VMEM((_RW, _HID), jnp.float32),
        pltpu.VMEM((_RW,), jnp.float32),
        pltpu.VMEM((_HID,), jnp.float32),
        pltpu.VMEM((_L,), jnp.float32),
        pltpu.SemaphoreType.DMA,
        pltpu.SemaphoreType.DMA,
        pltpu.SemaphoreType.DMA,
        pltpu.SemaphoreType.DMA,
    ],
    compiler_params=pltpu.CompilerParams(
        use_tc_tiling_on_sc=False, needs_layout_passes=False),
)


def _sc_cbow_body(ctx_hbm, emb_hbm, w_hbm, b_hbm, out_hbm,
                  idx_v, buf, pooled, acc_out, w_v, b_v, s0, s1, s2, s3):
    wid = lax.axis_index("s") * _NC + lax.axis_index("c")
    sems = (s0, s1, s2, s3)

    pltpu.sync_copy(ctx_hbm.at[wid], idx_v)
    pltpu.sync_copy(w_hbm, w_v)
    pltpu.sync_copy(b_hbm, b_v)

    def fire(r, k):
        pltpu.make_async_copy(
            emb_hbm.at[idx_v.at[r]], buf.at[k], sems[k],
        ).start()

    for k in range(_NBUF):
        fire(k, k)

    def chunk_body(r, carry):
        for k in range(_NBUF):
            @pl.when(lax.rem(r, _NBUF) == k)
            def _(k=k):
                pltpu.make_async_copy(
                    emb_hbm.at[idx_v.at[r]], buf.at[k], sems[k],
                ).wait()
                for rr in range(_PAIR):
                    def j_body(j, accs):
                        base = rr * _CTX + j
                        return tuple(
                            accs[c4] + buf[k, base, pl.ds(c4 * _L, _L)]
                            for c4 in range(_HID // _L)
                        )

                    accs = lax.fori_loop(
                        0, _CTX, j_body,
                        tuple(jnp.zeros((_L,), jnp.float32)
                              for _ in range(_HID // _L)),
                        unroll=5,
                    )
                    row = r * _PAIR + rr
                    for c4 in range(_HID // _L):
                        pooled[row, pl.ds(c4 * _L, _L)] = accs[c4]

                nxt = r + _NBUF
                @pl.when(nxt < _NCHK)
                def _():
                    fire(nxt, k)
        return carry

    lax.fori_loop(0, _NCHK, chunk_body, 0)

    lane = lax.iota(jnp.int32, _L)
    w4 = tuple(w_v[pl.ds(c4 * _L, _L)] for c4 in range(_HID // _L))
    bias = b_v[...]

    def grp_body(g, carry):
        acc = jnp.zeros((_L,), jnp.float32)
        for jj in range(_L):
            i = g * _L + jj
            s = jnp.zeros((_L,), jnp.float32)
            for c4 in range(_HID // _L):
                s = s + w4[c4] * pooled[i, pl.ds(c4 * _L, _L)]
            acc = jnp.where(lane == jj, jnp.sum(s), acc)
        acc_out[pl.ds(g * _L, _L)] = acc + bias
        return carry

    lax.fori_loop(0, _RW // _L, grp_body, 0)

    pltpu.sync_copy(acc_out, out_hbm.at[pl.ds(wid * _RW, _RW)])


_sc_cbow = pl.kernel(_sc_cbow_body, **_SC_KERNEL_KWARGS)


def kernel(context, emb, W, b):
    ctx3 = context.astype(jnp.int32).reshape(_NW, _NCHK, _REAL)
    ctx3 = jnp.pad(ctx3, ((0, 0), (0, 0), (0, _CHI - _REAL)))
    out = _sc_cbow(ctx3, emb, W.reshape(_HID), jnp.broadcast_to(b, (_L,)))
    return out.reshape(_B, 1)
